# baseline (speedup 1.0000x reference)
import functools

import jax
import jax.numpy as jnp
from jax.experimental import pallas as pl
from jax.experimental.pallas import tpu as pltpu

D_MODEL = 1024
D_SPACE = 64
N_FQK = 2048
N_FV = 2048
N_REL = 4096
N_VAL = 4096
N_USED = N_FQK + N_FV + N_REL + N_REL + N_VAL
B, S = 4, 2048
TOKENS = B * S
TBLK = 256
GRID = TOKENS // TBLK


def _logits_body(x_ref, w_ref, b_ref, embt_ref, out_ref):
    h = jax.lax.dot_general(
        x_ref[...].astype(jnp.bfloat16), w_ref[...].astype(jnp.bfloat16),
        (((1,), (0,)), ((), ())),
        preferred_element_type=jnp.float32) + b_ref[...]
    emb_t = embt_ref[...]
    inv = 1.0 / jnp.maximum(jnp.sqrt(jnp.sum(emb_t * emb_t, axis=0, keepdims=True)), 1e-12)
    emb_n = (emb_t * inv).astype(jnp.bfloat16)
    out_ref[...] = jax.lax.dot_general(
        h.astype(jnp.bfloat16), emb_n, (((1,), (0,)), ((), ())),
        preferred_element_type=jnp.float32)


@jax.jit
def _logits(x2d, W_proj, b_proj, emb_t):
    return pl.pallas_call(
        _logits_body,
        grid=(GRID,),
        in_specs=[
            pl.BlockSpec((TBLK, D_MODEL), lambda i: (i, 0)),
            pl.BlockSpec((D_MODEL, D_SPACE), lambda i: (0, 0)),
            pl.BlockSpec((1, D_SPACE), lambda i: (0, 0)),
            pl.BlockSpec((D_SPACE, N_USED), lambda i: (0, 0)),
        ],
        out_specs=pl.BlockSpec((TBLK, N_USED), lambda i: (i, 0)),
        out_shape=jax.ShapeDtypeStruct((TOKENS, N_USED), jnp.float32),
    )(x2d, W_proj, b_proj, emb_t)


def kernel(x, W_proj, b_proj, neuron_emb, neuron_emb_rel_k):
    emb_used = jnp.concatenate(
        [neuron_emb[:N_FQK + N_FV + N_REL + N_VAL], neuron_emb_rel_k], axis=0)
    emb_t = emb_used.T
    x2d = x.reshape(TOKENS, D_MODEL)
    logits = _logits(x2d, W_proj, b_proj.reshape(1, D_SPACE), emb_t)

    segs = [
        (0, N_FQK, 64),
        (N_FQK, N_FV, 32),
        (N_FQK + N_FV, N_REL, 64),
        (N_FQK + N_FV + N_REL + N_VAL, N_REL, 64),
        (N_FQK + N_FV + N_REL, N_VAL, 32),
    ]
    ws, idxs = [], []
    for off, n, k in segs:
        vals, idx = jax.lax.top_k(logits[:, off:off + n], k)
        ws.append(jax.nn.softmax(vals, axis=-1))
        idxs.append(idx)
    weights = jnp.concatenate(ws, axis=-1).reshape(B, S, 256)
    indices = jnp.concatenate(idxs, axis=-1).reshape(B, S, 256)
    return weights, indices


# speedup vs baseline: 8.9836x; 8.9836x over previous
import functools

import jax
import jax.numpy as jnp
from jax import lax
from jax.experimental import pallas as pl
from jax.experimental.pallas import tpu as pltpu
from jax.experimental.pallas import tpu_sc as plsc

D_MODEL = 1024
D_SPACE = 64
N_FQK = 2048
N_FV = 2048
N_REL = 4096
N_VAL = 4096
N_USED = N_FQK + N_FV + N_REL + N_VAL + N_REL
B, S = 4, 2048
TOKENS = B * S
TBLK = 256
GRID = TOKENS // TBLK

POOLS = (
    (0, N_FQK, 64, 0),
    (N_FQK, N_FV, 32, 64),
    (N_FQK + N_FV, N_REL, 64, 96),
    (N_FQK + N_FV + N_REL + N_VAL, N_REL, 64, 160),
    (N_FQK + N_FV + N_REL, N_VAL, 32, 224),
)
KTOT = 256
CAP = 384
NW = 32
TPW = TOKENS // NW
L = 16

NEG = float("-inf")


def _logits_body(x_ref, w_ref, b_ref, embt_ref, out_ref):
    h = lax.dot_general(
        x_ref[...].astype(jnp.bfloat16), w_ref[...].astype(jnp.bfloat16),
        (((1,), (0,)), ((), ())),
        preferred_element_type=jnp.float32) + b_ref[...]
    emb_t = embt_ref[...]
    inv = 1.0 / jnp.maximum(jnp.sqrt(jnp.sum(emb_t * emb_t, axis=0, keepdims=True)), 1e-12)
    emb_n = (emb_t * inv).astype(jnp.bfloat16)
    out_ref[...] = lax.dot_general(
        h.astype(jnp.bfloat16), emb_n, (((1,), (0,)), ((), ())),
        preferred_element_type=jnp.float32)


def _logits(x2d, W_proj, b_proj, emb_t):
    return pl.pallas_call(
        _logits_body,
        grid=(GRID,),
        in_specs=[
            pl.BlockSpec((TBLK, D_MODEL), lambda i: (i, 0)),
            pl.BlockSpec((D_MODEL, D_SPACE), lambda i: (0, 0)),
            pl.BlockSpec((1, D_SPACE), lambda i: (0, 0)),
            pl.BlockSpec((D_SPACE, N_USED), lambda i: (0, 0)),
        ],
        out_specs=pl.BlockSpec((TBLK, N_USED), lambda i: (i, 0)),
        out_shape=jax.ShapeDtypeStruct((TOKENS, N_USED), jnp.float32),
    )(x2d, W_proj, b_proj, emb_t)


def _srt16(k, v):
    return plsc.sort_key_val(k, v, descending=True)


def _cmpx(a, b):
    (ka, va), (kb, vb) = a, b
    m = ka >= kb
    kh = jnp.where(m, ka, kb)
    kl = jnp.where(m, kb, ka)
    vh = jnp.where(m, va, vb)
    vl = jnp.where(m, vb, va)
    return (kh, vh), (kl, vl)


def _flip(p):
    return (jnp.flip(p[0], 0), jnp.flip(p[1], 0))


def _bitonic_sort_run(run):
    n = len(run)
    if n == 1:
        return [_srt16(*run[0])]
    half = n // 2
    hs, ls = [], []
    for i in range(half):
        h, l = _cmpx(run[i], run[i + half])
        hs.append(h)
        ls.append(l)
    return _bitonic_sort_run(hs) + _bitonic_sort_run(ls)


def _merge_runs(a, b, out_n):
    n = len(a)
    brev = [_flip(b[n - 1 - i]) for i in range(n)]
    hs, ls = [], []
    for i in range(n):
        h, l = _cmpx(a[i], brev[i])
        hs.append(h)
        ls.append(l)
    out = _bitonic_sort_run(hs)
    if out_n > n:
        out = out + _bitonic_sort_run(ls)
    return out[:out_n]


def _sort_topk(keys, vals, out_n):
    runs = [[_srt16(k, v)] for k, v in zip(keys, vals)]
    while len(runs) > 1:
        nxt = []
        for i in range(0, len(runs) - 1, 2):
            n = len(runs[i])
            keep = min(2 * n, max(out_n, n))
            nxt.append(_merge_runs(runs[i], runs[i + 1], keep))
        if len(runs) % 2:
            nxt.append(runs[-1])
        runs = nxt
    return runs[0][:out_n]


def _sel_body(lg_hbm, w_hbm, i_hbm, lg_v, ck_v, ci_v, wrow_v, irow_v):
    nc = 2
    wid = lax.axis_index("s") * nc + lax.axis_index("c")
    lane = lax.iota(jnp.int32, L)
    neg = jnp.full((L,), NEG, jnp.float32)

    def token_body(t, carry):
        tok = wid * TPW + t
        pltpu.sync_copy(lg_hbm.at[tok], lg_v)

        for (c_off, n, k, o_off) in POOLS:
            m = 4 if k == 64 else 2
            nv = n // L

            def pa(i, rs):
                base = c_off + i * (4 * L)
                v0 = lg_v[pl.ds(base, L)]
                v1 = lg_v[pl.ds(base + L, L)]
                v2 = lg_v[pl.ds(base + 2 * L, L)]
                v3 = lg_v[pl.ds(base + 3 * L, L)]
                v = jnp.maximum(jnp.maximum(v0, v1), jnp.maximum(v2, v3))
                out = []
                for r in rs:
                    hi = jnp.maximum(r, v)
                    v = jnp.minimum(r, v)
                    out.append(hi)
                return tuple(out)

            rs = lax.fori_loop(0, nv // 4, pa, tuple(neg for _ in range(m)))
            rasc, _ = plsc.sort_key_val(rs[m - 1], rs[m - 1])
            t0 = rasc[0]

            for j in range(CAP // L):
                ck_v[pl.ds(j * L, L)] = neg

            def pb(i, off):
                v = lg_v[pl.ds(c_off + i * L, L)]
                msk = v >= t0
                offc = jnp.minimum(off, CAP - L)
                plsc.store_compressed(ck_v.at[pl.ds(offc, L)], v, mask=msk)
                plsc.store_compressed(
                    ci_v.at[pl.ds(offc, L)], lane + i * L, mask=msk)
                return off + plsc.all_reduce_population_count(msk)[0]

            lax.fori_loop(0, nv, pb, jnp.int32(0))

            keys = [ck_v[pl.ds(j * L, L)] for j in range(CAP // L)]
            vals = [ci_v[pl.ds(j * L, L)] for j in range(CAP // L)]
            top = _sort_topk(keys, vals, k // L)

            mx = top[0][0][0]
            es = [jnp.exp(kk - mx) for kk, _ in top]
            s = es[0] if len(es) == 1 else functools.reduce(jnp.add, es)
            tot = plsc.cumsum(s)[L - 1]
            for j, (e, (_, iv)) in enumerate(zip(es, top)):
                wrow_v[pl.ds(o_off + j * L, L)] = e / tot
                irow_v[pl.ds(o_off + j * L, L)] = iv

        pltpu.sync_copy(wrow_v, w_hbm.at[tok])
        pltpu.sync_copy(irow_v, i_hbm.at[tok])
        return carry

    lax.fori_loop(0, TPW, token_body, jnp.int32(0))


def _select(logits):
    mesh = plsc.VectorSubcoreMesh(
        core_axis_name="c", subcore_axis_name="s", num_cores=2, num_subcores=16)
    return pl.kernel(
        _sel_body,
        out_type=(
            jax.ShapeDtypeStruct((TOKENS, KTOT), jnp.float32),
            jax.ShapeDtypeStruct((TOKENS, KTOT), jnp.int32),
        ),
        mesh=mesh,
        compiler_params=pltpu.CompilerParams(needs_layout_passes=False),
        scratch_types=[
            pltpu.VMEM((N_USED,), jnp.float32),
            pltpu.VMEM((CAP,), jnp.float32),
            pltpu.VMEM((CAP,), jnp.int32),
            pltpu.VMEM((KTOT,), jnp.float32),
            pltpu.VMEM((KTOT,), jnp.int32),
        ],
    )(logits)


def kernel(x, W_proj, b_proj, neuron_emb, neuron_emb_rel_k):
    emb_used = jnp.concatenate(
        [neuron_emb[:N_FQK + N_FV + N_REL + N_VAL], neuron_emb_rel_k], axis=0)
    emb_t = emb_used.T
    x2d = x.reshape(TOKENS, D_MODEL)
    logits = _logits(x2d, W_proj, b_proj.reshape(1, D_SPACE), emb_t)
    w2d, i2d = _select(logits)
    return w2d.reshape(B, S, KTOT), i2d.reshape(B, S, KTOT)
